# baseline (speedup 1.0000x reference)
import functools

import jax
import jax.numpy as jnp
from jax import lax
from jax.experimental import pallas as pl
from jax.experimental.pallas import tpu as pltpu
from jax.experimental.pallas import tpu_sc as plsc

STRIDE = 4
PATCH = 31
B = 4
A = 8
C = 256
H = 127
W = 127
HC = 25
WC = 25
L = 16
CH = C // 128
NROW = 4
NCOL = 2
NSLOT = NROW * NCOL
NJ = (PATCH + NCOL - 1) // NCOL


def _sc_body(full_rows, corr_rows, anc_ref, out_rows,
             anc_v, crow_v, wbuf_v, gbuf_v,
             csem, *slot_sems):
    gsems = slot_sems[:NSLOT]
    osems = slot_sems[NSLOT:]

    wid = lax.axis_index("s") * 2 + lax.axis_index("c")
    b = wid // A
    ig = wid % A
    i0 = ig * NROW

    lanes = lax.iota(jnp.int32, L)
    av = lanes & 7
    chv = lanes >> 3

    pltpu.sync_copy(anc_ref, anc_v)
    ay = plsc.load_gather(anc_v, [(b * A + av) * 2])
    ax = plsc.load_gather(anc_v, [(b * A + av) * 2 + 1])

    cidx = ((ay * WC + ax) * CH + chv) * B + b
    pltpu.async_copy(corr_rows.at[cidx], crow_v, csem).wait()

    for a in range(A):
        m = jnp.maximum(crow_v[a, pl.ds(0, L)], crow_v[a + A, pl.ds(0, L)])
        for k in range(1, 128 // L):
            m = jnp.maximum(m, crow_v[a, pl.ds(k * L, L)])
            m = jnp.maximum(m, crow_v[a + A, pl.ds(k * L, L)])
        mm = jnp.max(m)
        s = jnp.zeros((L,), jnp.float32)
        for row in (a, a + A):
            for k in range(128 // L):
                e = jnp.exp(crow_v[row, pl.ds(k * L, L)] - mm)
                wbuf_v[row, pl.ds(k * L, L)] = e
                s = s + e
        invv = 1.0 / jnp.full((L,), jnp.sum(s), dtype=jnp.float32)
        for row in (a, a + A):
            for k in range(128 // L):
                wbuf_v[row, pl.ds(k * L, L)] = (
                    wbuf_v[row, pl.ds(k * L, L)] * invv)

    hbase = ay * STRIDE
    wbase = ax * STRIDE

    def gather_idx(i, j):
        hv = jnp.minimum(hbase + i, H - 1)
        wv = jnp.minimum(wbase + j, W - 1)
        return ((hv * W + wv) * CH + chv) * B + b

    def out_base(i, j):
        return ((b * PATCH + i) * PATCH + j) * L

    def scale_all():
        def l_body(l, carry):
            wrow = [wbuf_v[l, pl.ds(k * L, L)] for k in range(128 // L)]
            for s in range(NSLOT):
                for k in range(128 // L):
                    gbuf_v[s, l, pl.ds(k * L, L)] = (
                        gbuf_v[s, l, pl.ds(k * L, L)] * wrow[k])
            return carry
        lax.fori_loop(0, L, l_body, 0)

    def jj_body(jj, carry):
        for s in range(NSLOT):
            i = i0 + s % NROW
            j = jj * NCOL + s // NROW
            @pl.when(jnp.logical_and(jj > 0, i < PATCH))
            def _wait_prev_out():
                pltpu.make_async_copy(
                    gbuf_v.at[s], out_rows.at[pl.ds(0, L)],
                    osems[s]).wait()

            pltpu.async_copy(full_rows.at[gather_idx(i, j)],
                             gbuf_v.at[s], gsems[s])
        for s in range(NSLOT):
            pltpu.make_async_copy(full_rows.at[pl.ds(0, L)],
                                  gbuf_v.at[s], gsems[s]).wait()
        scale_all()
        for s in range(NSLOT):
            i = i0 + s % NROW
            j = jj * NCOL + s // NROW
            valid = jnp.logical_and(i < PATCH, j < PATCH)

            @pl.when(valid)
            def _store_out():
                pltpu.async_copy(
                    gbuf_v.at[s],
                    out_rows.at[pl.ds(out_base(i, j), L)], osems[s])
        return carry

    lax.fori_loop(0, NJ, jj_body, 0)

    for s in range(NSLOT):
        i = i0 + s % NROW
        j = (NJ - 1) * NCOL + s // NROW
        valid = jnp.logical_and(i < PATCH, j < PATCH)

        @pl.when(valid)
        def _drain():
            pltpu.make_async_copy(
                gbuf_v.at[s], out_rows.at[pl.ds(0, L)], osems[s]).wait()


@jax.jit
def _run(full_feature, corr_feature, anchor_flat):
    full_rows = full_feature.reshape(B, CH, 128, H, W).transpose(
        3, 4, 1, 0, 2).reshape(H * W * CH * B, 128)
    corr_rows = corr_feature.reshape(B, CH, 128, HC, WC).transpose(
        3, 4, 1, 0, 2).reshape(HC * WC * CH * B, 128)

    mesh = plsc.VectorSubcoreMesh(core_axis_name="c", subcore_axis_name="s")
    fn = pl.kernel(
        _sc_body,
        out_type=jax.ShapeDtypeStruct((B * PATCH * PATCH * L, 128),
                                      jnp.float32),
        mesh=mesh,
        compiler_params=pltpu.CompilerParams(
            use_tc_tiling_on_sc=False, needs_layout_passes=False),
        scratch_types=[
            pltpu.VMEM((2 * B * A,), jnp.int32),
            pltpu.VMEM((L, 128), jnp.float32),
            pltpu.VMEM((L, 128), jnp.float32),
            pltpu.VMEM((NSLOT, L, 128), jnp.float32),
        ] + [pltpu.SemaphoreType.DMA] * (1 + 2 * NSLOT),
    )
    out_rows = fn(full_rows, corr_rows, anchor_flat)
    return out_rows.reshape(B, PATCH, PATCH, CH, A, 128).transpose(
        0, 4, 3, 5, 1, 2).reshape(B, A, C, PATCH, PATCH)


def kernel(full_feature, corr_feature, anchor):
    anchor_flat = anchor.reshape(-1).astype(jnp.int32)
    return _run(full_feature, corr_feature, anchor_flat)
